# Initial kernel scaffold; baseline (speedup 1.0000x reference)
import jax
import jax.numpy as jnp
from jax.experimental import pallas as pl


def kernel(hidden_states, router_w, w_gate, w_up, w_down):
    raise NotImplementedError("write your pallas kernel here")


# speedup vs baseline: 1.1069x; 1.1069x over previous
import functools

import jax
import jax.numpy as jnp
from jax import lax
from jax.experimental import pallas as pl
from jax.experimental.pallas import tpu as pltpu
from jax.experimental.pallas import tpu_sc as plsc

B, S, D, E, K, F = 1, 2048, 1024, 8, 2, 1408
T = B * S
BT = 256
NB = 24
P = NB * BT

NW = 32
G_PER_W = P // NW
G_CH = 96
C_PER_W = T // NW
C_CH = 16


def _router_body(x_ref, rw_ref, e0_ref, e1_ref, w0_ref, w1_ref):
    x = x_ref[...]
    rw = rw_ref[...]
    logits = jnp.dot(x, rw, preferred_element_type=jnp.float32)
    ids = lax.broadcasted_iota(jnp.int32, (T, E), 1)
    m0 = jnp.max(logits, axis=1)
    e0 = jnp.min(jnp.where(logits == m0[:, None], ids, E), axis=1)
    masked = jnp.where(ids == e0[:, None], -jnp.inf, logits)
    m1 = jnp.max(masked, axis=1)
    e1 = jnp.min(jnp.where(masked == m1[:, None], ids, E), axis=1)
    z = jnp.exp(m1 - m0)
    w0 = 1.0 / (1.0 + z)
    e0_ref[...] = e0
    e1_ref[...] = e1
    w0_ref[...] = w0
    w1_ref[...] = z * w0


def _router(x, rw):
    return pl.pallas_call(
        _router_body,
        out_shape=(
            jax.ShapeDtypeStruct((T,), jnp.int32),
            jax.ShapeDtypeStruct((T,), jnp.int32),
            jax.ShapeDtypeStruct((T,), jnp.float32),
            jax.ShapeDtypeStruct((T,), jnp.float32),
        ),
    )(x, rw)


def _routing_meta(e0, e1, w0, w1):
    ep = jnp.stack([e0, e1], axis=1).reshape(-1)
    wf = jnp.stack([w0, w1], axis=1).reshape(-1)
    oh = (ep[:, None] == jnp.arange(E, dtype=jnp.int32)[None, :]).astype(jnp.int32)
    counts = jnp.sum(oh, axis=0)
    rank = jnp.cumsum(oh, axis=0) - oh
    rank_p = jnp.sum(rank * oh, axis=1)
    aligned = ((counts + BT - 1) // BT) * BT
    starts = jnp.concatenate([jnp.zeros(1, jnp.int32), jnp.cumsum(aligned)])[:E]
    slot = starts[ep] + rank_p
    pair_tok = jnp.repeat(jnp.arange(T, dtype=jnp.int32), K)
    src_tok = jnp.zeros((P,), jnp.int32).at[slot].set(pair_tok)
    ws = jnp.zeros((P,), jnp.float32).at[slot].set(wf)
    pos0, pos1 = slot[0::2], slot[1::2]
    ends = starts + aligned
    base = jnp.arange(NB, dtype=jnp.int32) * BT
    be = jnp.minimum(jnp.sum((base[:, None] >= ends[None, :]).astype(jnp.int32), axis=1), E - 1)
    bv = (base < ends[E - 1]).astype(jnp.int32)
    return src_tok, ws.reshape(NB, 1, BT), pos0, pos1, be, bv


_SC_MESH = plsc.VectorSubcoreMesh(core_axis_name="c", subcore_axis_name="s")


@functools.partial(
    pl.kernel,
    out_type=jax.ShapeDtypeStruct((P, D), jnp.float32),
    mesh=_SC_MESH,
    scratch_types=[
        pltpu.VMEM((G_CH,), jnp.int32),
        pltpu.VMEM((G_CH, D), jnp.float32),
        pltpu.SemaphoreType.DMA,
    ],
)
def _gather_rows(x_hbm, idx_hbm, out_hbm, idx_v, rows_v, sem):
    wid = lax.axis_index("s") * 2 + lax.axis_index("c")
    base = wid * G_PER_W
    for c in range(G_PER_W // G_CH):
        off = base + c * G_CH
        pltpu.sync_copy(idx_hbm.at[pl.ds(off, G_CH)], idx_v)
        pltpu.async_copy(x_hbm.at[idx_v], rows_v, sem).wait()
        pltpu.sync_copy(rows_v, out_hbm.at[pl.ds(off, G_CH)])


def _mlp_body(be_ref, bv_ref, xs_ref, wg_ref, wu_ref, wd_ref, ws_ref, out_ref):
    b = pl.program_id(0)

    @pl.when(bv_ref[b] != 0)
    def _compute():
        x = xs_ref[...]
        wg = wg_ref[0]
        wu = wu_ref[0]
        wd = wd_ref[0]
        g = jnp.dot(x, wg, preferred_element_type=jnp.float32)
        u = jnp.dot(x, wu, preferred_element_type=jnp.float32)
        h = (g * jax.nn.sigmoid(g)) * u
        h = h * ws_ref[0, 0][:, None]
        out_ref[...] = jnp.dot(h, wd, preferred_element_type=jnp.float32)

    @pl.when(bv_ref[b] == 0)
    def _skip():
        out_ref[...] = jnp.zeros_like(out_ref)


def _grouped_mlp(be, bv, xs, w_gate, w_up, w_down, ws2d):
    grid_spec = pltpu.PrefetchScalarGridSpec(
        num_scalar_prefetch=2,
        grid=(NB,),
        in_specs=[
            pl.BlockSpec((BT, D), lambda b, be, bv: (b, 0)),
            pl.BlockSpec((1, D, F), lambda b, be, bv: (be[b], 0, 0)),
            pl.BlockSpec((1, D, F), lambda b, be, bv: (be[b], 0, 0)),
            pl.BlockSpec((1, F, D), lambda b, be, bv: (be[b], 0, 0)),
            pl.BlockSpec((1, 1, BT), lambda b, be, bv: (b, 0, 0)),
        ],
        out_specs=pl.BlockSpec((BT, D), lambda b, be, bv: (b, 0)),
    )
    return pl.pallas_call(
        _mlp_body,
        grid_spec=grid_spec,
        out_shape=jax.ShapeDtypeStruct((P, D), jnp.float32),
        compiler_params=pltpu.CompilerParams(dimension_semantics=("arbitrary",)),
    )(be, bv, xs, w_gate, w_up, w_down, ws2d)


@functools.partial(
    pl.kernel,
    out_type=jax.ShapeDtypeStruct((T, D), jnp.float32),
    mesh=_SC_MESH,
    scratch_types=[
        pltpu.VMEM((C_CH,), jnp.int32),
        pltpu.VMEM((C_CH,), jnp.int32),
        pltpu.VMEM((C_CH, D), jnp.float32),
        pltpu.VMEM((C_CH, D), jnp.float32),
        pltpu.VMEM((C_CH, D), jnp.float32),
        pltpu.SemaphoreType.DMA,
        pltpu.SemaphoreType.DMA,
    ],
)
def _combine(x_hbm, ys_hbm, pos0_hbm, pos1_hbm, out_hbm,
             p0_v, p1_v, id_v, a_v, b_v, sem_a, sem_b):
    wid = lax.axis_index("s") * 2 + lax.axis_index("c")
    base = wid * C_PER_W
    for c in range(C_PER_W // C_CH):
        off = base + c * C_CH
        pltpu.sync_copy(pos0_hbm.at[pl.ds(off, C_CH)], p0_v)
        pltpu.sync_copy(pos1_hbm.at[pl.ds(off, C_CH)], p1_v)
        cp_a = pltpu.async_copy(ys_hbm.at[p0_v], a_v, sem_a)
        cp_b = pltpu.async_copy(ys_hbm.at[p1_v], b_v, sem_b)
        pltpu.sync_copy(x_hbm.at[pl.ds(off, C_CH)], id_v)
        cp_a.wait()
        cp_b.wait()

        def _row(i, _):
            def _col(j, _):
                s = pl.ds(j * 16, 16)
                id_v[i, s] = id_v[i, s] + a_v[i, s] + b_v[i, s]
                return 0
            lax.fori_loop(0, D // 16, _col, 0)
            return 0

        lax.fori_loop(0, C_CH, _row, 0)
        pltpu.sync_copy(id_v, out_hbm.at[pl.ds(off, C_CH)])


def kernel(hidden_states, router_w, w_gate, w_up, w_down):
    x = hidden_states.reshape(T, D)
    e0, e1, w0, w1 = _router(x, router_w)
    src_tok, ws2d, pos0, pos1, be, bv = _routing_meta(e0, e1, w0, w1)
    xs = _gather_rows(x, src_tok)
    ys = _grouped_mlp(be, bv, xs, w_gate, w_up, w_down, ws2d)
    y = _combine(x, ys, pos0, pos1)
    return y.reshape(B, S, D)
